# baseline (speedup 1.0000x reference)
import functools

import jax
import jax.numpy as jnp
from jax import lax
from jax.experimental import pallas as pl
from jax.experimental.pallas import tpu as pltpu
from jax.experimental.pallas import tpu_sc as plsc

VOCAB = 100000
EMBED_DIM = 300
BATCH = 1024
SEQ = 200

NUM_WORKERS = 32
PLANES_PER_WORKER = (EMBED_DIM + NUM_WORKERS - 1) // NUM_WORKERS
ROWS_PER_BLOCK = 8
NUM_BLOCKS = SEQ // ROWS_PER_BLOCK
LANES = 16
VREGS_PER_ROW = BATCH // LANES

_mesh = plsc.VectorSubcoreMesh(core_axis_name="c", subcore_axis_name="s")


@functools.partial(
    pl.kernel,
    mesh=_mesh,
    out_type=jax.ShapeDtypeStruct((EMBED_DIM, SEQ, BATCH), jnp.float32),
    scratch_types=[
        pltpu.VMEM((VOCAB,), jnp.float32),
        pltpu.VMEM((ROWS_PER_BLOCK, BATCH), jnp.int32),
        pltpu.VMEM((ROWS_PER_BLOCK, BATCH), jnp.float32),
    ],
    compiler_params=pltpu.CompilerParams(use_tc_tiling_on_sc=True,
                                         needs_layout_passes=False),
)
def _gather_kernel(embT_hbm, wsT_hbm, outT_hbm, col_v, idx_v, out_v):
    wid = lax.axis_index("s") * 2 + lax.axis_index("c")

    def plane_body(t, carry):
        d = wid + NUM_WORKERS * t

        @pl.when(d < EMBED_DIM)
        def _():
            pltpu.sync_copy(embT_hbm.at[d], col_v)

            def block_body(g, carry2):
                row0 = pl.multiple_of(g * ROWS_PER_BLOCK, ROWS_PER_BLOCK)
                pltpu.sync_copy(wsT_hbm.at[pl.ds(row0, ROWS_PER_BLOCK)], idx_v)

                def row_body(r, carry3):
                    for k in range(VREGS_PER_ROW):
                        iv = idx_v[r, pl.ds(k * LANES, LANES)]
                        out_v[r, pl.ds(k * LANES, LANES)] = (
                            plsc.load_gather(col_v, [iv]))
                    return carry3

                lax.fori_loop(0, ROWS_PER_BLOCK, row_body, 0)
                pltpu.sync_copy(out_v,
                                outT_hbm.at[d, pl.ds(row0, ROWS_PER_BLOCK)])
                return carry2

            lax.fori_loop(0, NUM_BLOCKS, block_body, 0)

        return carry

    lax.fori_loop(0, PLANES_PER_WORKER, plane_body, 0)


def kernel(word_sequences, embedding):
    wsT = word_sequences.T
    embT = embedding.T
    outT = _gather_kernel(embT, wsT)
    return outT.transpose(2, 1, 0)


# speedup vs baseline: 1.3637x; 1.3637x over previous
import functools

import jax
import jax.numpy as jnp
from jax import lax
from jax.experimental import pallas as pl
from jax.experimental.pallas import tpu as pltpu
from jax.experimental.pallas import tpu_sc as plsc

VOCAB = 100000
EMBED_DIM = 300
BATCH = 1024
SEQ = 200

NUM_WORKERS = 32
PLANES_PER_WORKER = (EMBED_DIM + NUM_WORKERS - 1) // NUM_WORKERS
ROWS = 8
NUM_BLOCKS = SEQ // ROWS
LANES = 16
VREGS_PER_ROW = BATCH // LANES

_mesh = plsc.VectorSubcoreMesh(core_axis_name="c", subcore_axis_name="s")


@functools.partial(
    pl.kernel,
    mesh=_mesh,
    out_type=jax.ShapeDtypeStruct((EMBED_DIM, SEQ, BATCH), jnp.float32),
    scratch_types=[
        pltpu.VMEM((VOCAB,), jnp.float32),
        pltpu.VMEM((ROWS, BATCH), jnp.int32),
        pltpu.VMEM((ROWS, BATCH), jnp.int32),
        pltpu.VMEM((ROWS, BATCH), jnp.float32),
        pltpu.SemaphoreType.DMA,
        pltpu.SemaphoreType.DMA,
        pltpu.SemaphoreType.DMA,
    ],
    compiler_params=pltpu.CompilerParams(use_tc_tiling_on_sc=True,
                                         needs_layout_passes=False),
)
def _gather_kernel(embT_hbm, wsT_hbm, outT_hbm, col_v, idx0, idx1, out_v,
                   sem_i0, sem_i1, sem_o):
    wid = lax.axis_index("s") * 2 + lax.axis_index("c")

    def gather_block(cur_idx, g, d):
        def row_body(r, carry):
            for k in range(VREGS_PER_ROW):
                iv = cur_idx[r, pl.ds(k * LANES, LANES)]
                out_v[r, pl.ds(k * LANES, LANES)] = (
                    plsc.load_gather(col_v, [iv]))
            return carry

        lax.fori_loop(0, ROWS, row_body, 0)
        pltpu.async_copy(out_v, outT_hbm.at[d, pl.ds(g * ROWS, ROWS)], sem_o)

    def do_block(g, d, cur_idx, cur_sem, nxt_idx, nxt_sem):
        pltpu.make_async_copy(
            wsT_hbm.at[pl.ds(g * ROWS, ROWS)], cur_idx, cur_sem).wait()

        @pl.when(g + 1 < NUM_BLOCKS)
        def _():
            pltpu.async_copy(
                wsT_hbm.at[pl.ds((g + 1) * ROWS, ROWS)], nxt_idx, nxt_sem)

        @pl.when(g > 0)
        def _():
            pltpu.make_async_copy(
                outT_hbm.at[d, pl.ds(0, ROWS)], out_v, sem_o).wait()

        gather_block(cur_idx, g, d)

    def plane_body(t, carry):
        d = wid + NUM_WORKERS * t

        @pl.when(d < EMBED_DIM)
        def _():
            pltpu.async_copy(wsT_hbm.at[pl.ds(0, ROWS)], idx0, sem_i0)
            pltpu.sync_copy(embT_hbm.at[d], col_v)

            def block_body(g, carry2):
                @pl.when(lax.rem(g, 2) == 0)
                def _():
                    do_block(g, d, idx0, sem_i0, idx1, sem_i1)

                @pl.when(lax.rem(g, 2) == 1)
                def _():
                    do_block(g, d, idx1, sem_i1, idx0, sem_i0)

                return carry2

            lax.fori_loop(0, NUM_BLOCKS, block_body, 0)
            pltpu.make_async_copy(
                outT_hbm.at[d, pl.ds(0, ROWS)], out_v, sem_o).wait()

        return carry

    lax.fori_loop(0, PLANES_PER_WORKER, plane_body, 0)


def kernel(word_sequences, embedding):
    wsT = word_sequences.T
    embT = embedding.T
    outT = _gather_kernel(embT, wsT)
    return outT.transpose(2, 1, 0)


# speedup vs baseline: 2.4435x; 1.7918x over previous
import functools

import jax
import jax.numpy as jnp
from jax import lax
from jax.experimental import pallas as pl
from jax.experimental.pallas import tpu as pltpu
from jax.experimental.pallas import tpu_sc as plsc

VOCAB = 100000
EMBED_DIM = 300
BATCH = 1024
SEQ = 200

NUM_WORKERS = 32
PLANES_PER_WORKER = (EMBED_DIM + NUM_WORKERS - 1) // NUM_WORKERS
ROWS = 8
NUM_BLOCKS = SEQ // ROWS
LANES = 16
VREGS_PER_ROW = BATCH // LANES

_mesh = plsc.VectorSubcoreMesh(core_axis_name="c", subcore_axis_name="s")


@functools.partial(
    pl.kernel,
    mesh=_mesh,
    out_type=jax.ShapeDtypeStruct((EMBED_DIM, SEQ, BATCH), jnp.float32),
    scratch_types=[
        pltpu.VMEM((VOCAB,), jnp.float32),
        pltpu.VMEM((ROWS, BATCH), jnp.int32),
        pltpu.VMEM((ROWS, BATCH), jnp.int32),
        pltpu.VMEM((ROWS, BATCH), jnp.float32),
        pltpu.SemaphoreType.DMA,
        pltpu.SemaphoreType.DMA,
        pltpu.SemaphoreType.DMA,
    ],
    compiler_params=pltpu.CompilerParams(use_tc_tiling_on_sc=True,
                                         needs_layout_passes=False),
)
def _gather_kernel(embT_hbm, wsT_hbm, outT_hbm, col_v, idx0, idx1, out_v,
                   sem_i0, sem_i1, sem_o):
    wid = lax.axis_index("s") * 2 + lax.axis_index("c")

    def gather_block(cur_idx, g, d):
        def row_body(r, carry):
            @plsc.parallel_loop(0, BATCH, step=LANES, unroll=8)
            def _(c):
                iv = cur_idx[r, pl.ds(c, LANES)]
                out_v[r, pl.ds(c, LANES)] = plsc.load_gather(col_v, [iv])

            return carry

        lax.fori_loop(0, ROWS, row_body, 0)
        pltpu.async_copy(out_v, outT_hbm.at[d, pl.ds(g * ROWS, ROWS)], sem_o)

    def do_block(g, d, cur_idx, cur_sem, nxt_idx, nxt_sem):
        pltpu.make_async_copy(
            wsT_hbm.at[pl.ds(g * ROWS, ROWS)], cur_idx, cur_sem).wait()

        @pl.when(g + 1 < NUM_BLOCKS)
        def _():
            pltpu.async_copy(
                wsT_hbm.at[pl.ds((g + 1) * ROWS, ROWS)], nxt_idx, nxt_sem)

        @pl.when(g > 0)
        def _():
            pltpu.make_async_copy(
                outT_hbm.at[d, pl.ds(0, ROWS)], out_v, sem_o).wait()

        gather_block(cur_idx, g, d)

    def plane_body(t, carry):
        d = wid + NUM_WORKERS * t

        @pl.when(d < EMBED_DIM)
        def _():
            pltpu.async_copy(wsT_hbm.at[pl.ds(0, ROWS)], idx0, sem_i0)
            pltpu.sync_copy(embT_hbm.at[d], col_v)

            def block_body(g, carry2):
                @pl.when(lax.rem(g, 2) == 0)
                def _():
                    do_block(g, d, idx0, sem_i0, idx1, sem_i1)

                @pl.when(lax.rem(g, 2) == 1)
                def _():
                    do_block(g, d, idx1, sem_i1, idx0, sem_i0)

                return carry2

            lax.fori_loop(0, NUM_BLOCKS, block_body, 0)
            pltpu.make_async_copy(
                outT_hbm.at[d, pl.ds(0, ROWS)], out_v, sem_o).wait()

        return carry

    lax.fori_loop(0, PLANES_PER_WORKER, plane_body, 0)


def kernel(word_sequences, embedding):
    wsT = word_sequences.T
    embT = embedding.T
    outT = _gather_kernel(embT, wsT)
    return outT.transpose(2, 1, 0)


# speedup vs baseline: 2.4444x; 1.0004x over previous
import functools

import jax
import jax.numpy as jnp
from jax import lax
from jax.experimental import pallas as pl
from jax.experimental.pallas import tpu as pltpu
from jax.experimental.pallas import tpu_sc as plsc

VOCAB = 100000
EMBED_DIM = 300
BATCH = 1024
SEQ = 200

NUM_WORKERS = 32
PLANES_PER_WORKER = (EMBED_DIM + NUM_WORKERS - 1) // NUM_WORKERS
ROWS = 8
NUM_BLOCKS = SEQ // ROWS
LANES = 16
VREGS_PER_ROW = BATCH // LANES

_mesh = plsc.VectorSubcoreMesh(core_axis_name="c", subcore_axis_name="s")


@functools.partial(
    pl.kernel,
    mesh=_mesh,
    out_type=jax.ShapeDtypeStruct((EMBED_DIM, SEQ, BATCH), jnp.float32),
    scratch_types=[
        pltpu.VMEM((VOCAB,), jnp.float32),
        pltpu.VMEM((ROWS, BATCH), jnp.int32),
        pltpu.VMEM((ROWS, BATCH), jnp.int32),
        pltpu.VMEM((ROWS, BATCH), jnp.float32),
        pltpu.SemaphoreType.DMA,
        pltpu.SemaphoreType.DMA,
        pltpu.SemaphoreType.DMA,
    ],
    compiler_params=pltpu.CompilerParams(use_tc_tiling_on_sc=True,
                                         needs_layout_passes=False),
)
def _gather_kernel(embT_hbm, wsT_hbm, outT_hbm, col_v, idx0, idx1, out_v,
                   sem_i0, sem_i1, sem_o):
    wid = lax.axis_index("s") * 2 + lax.axis_index("c")

    def gather_block(cur_idx, g, d):
        def row_body(r, carry):
            @plsc.parallel_loop(0, BATCH, step=LANES, unroll=16)
            def _(c):
                iv = cur_idx[r, pl.ds(c, LANES)]
                out_v[r, pl.ds(c, LANES)] = plsc.load_gather(col_v, [iv])

            return carry

        lax.fori_loop(0, ROWS, row_body, 0)
        pltpu.async_copy(out_v, outT_hbm.at[d, pl.ds(g * ROWS, ROWS)], sem_o)

    def do_block(g, d, cur_idx, cur_sem, nxt_idx, nxt_sem):
        pltpu.make_async_copy(
            wsT_hbm.at[pl.ds(g * ROWS, ROWS)], cur_idx, cur_sem).wait()

        @pl.when(g + 1 < NUM_BLOCKS)
        def _():
            pltpu.async_copy(
                wsT_hbm.at[pl.ds((g + 1) * ROWS, ROWS)], nxt_idx, nxt_sem)

        @pl.when(g > 0)
        def _():
            pltpu.make_async_copy(
                outT_hbm.at[d, pl.ds(0, ROWS)], out_v, sem_o).wait()

        gather_block(cur_idx, g, d)

    def plane_body(t, carry):
        d = wid + NUM_WORKERS * t

        @pl.when(d < EMBED_DIM)
        def _():
            pltpu.async_copy(wsT_hbm.at[pl.ds(0, ROWS)], idx0, sem_i0)
            pltpu.sync_copy(embT_hbm.at[d], col_v)

            def block_body(g, carry2):
                @pl.when(lax.rem(g, 2) == 0)
                def _():
                    do_block(g, d, idx0, sem_i0, idx1, sem_i1)

                @pl.when(lax.rem(g, 2) == 1)
                def _():
                    do_block(g, d, idx1, sem_i1, idx0, sem_i0)

                return carry2

            lax.fori_loop(0, NUM_BLOCKS, block_body, 0)
            pltpu.make_async_copy(
                outT_hbm.at[d, pl.ds(0, ROWS)], out_v, sem_o).wait()

        return carry

    lax.fori_loop(0, PLANES_PER_WORKER, plane_body, 0)


def kernel(word_sequences, embedding):
    wsT = word_sequences.T
    embT = embedding.T
    outT = _gather_kernel(embT, wsT)
    return outT.transpose(2, 1, 0)
